# baseline (speedup 1.0000x reference)
import functools

import jax
import jax.numpy as jnp
from jax import lax
from jax.experimental import pallas as pl
from jax.experimental.pallas import tpu as pltpu
from jax.experimental.pallas import tpu_sc as plsc

_NC = 2
_NS = 16
_NW = _NC * _NS
_GRP = 128
_K = 8
_ZR = 64
_SPLIT = 0.5


@functools.lru_cache(maxsize=None)
def _sc_spmm(n_pad, e_pad, d, g0, g1):
    assert e_pad == _NS * (g0 + g1) * _GRP
    assert g0 % _K == 0 and g1 % _K == 0
    rows_per_tile = n_pad // _NS
    mesh = plsc.VectorSubcoreMesh(core_axis_name="c", subcore_axis_name="s")

    def body(src_h, dst_h, h_h, out_h,
             idx_s0, idx_s1, idx_d0, idx_d1, rows_a, rows_b, zbuf, acc,
             sem, sem_i):
        c = lax.axis_index("c")
        s = lax.axis_index("s")
        idx_s = [idx_s0, idx_s1]
        idx_d = [idx_d0, idx_d1]

        @pl.loop(0, _ZR)
        def _zfill(i):
            for j in range(d // 16):
                zbuf[i, pl.ds(j * 16, 16)] = jnp.zeros((16,), jnp.float32)
        zb = s * rows_per_tile

        @pl.loop(0, rows_per_tile // _ZR)
        def _zcopy(i):
            pltpu.sync_copy(zbuf, acc.at[pl.ds(zb + i * _ZR, _ZR)])

        plsc.subcore_barrier()

        def run(gbase, n_chunks):
            pltpu.sync_copy(src_h.at[pl.ds(gbase, _K)], idx_s[0])
            pltpu.sync_copy(dst_h.at[pl.ds(gbase, _K)], idx_d[0])
            pltpu.async_copy(h_h.at[idx_s[0].at[0]], rows_a, sem)

            for i in range(n_chunks):
                b, nb = i % 2, (i + 1) % 2
                last = i + 1 == n_chunks
                if not last:
                    nxt = gbase + (i + 1) * _K
                    pltpu.async_copy(src_h.at[pl.ds(nxt, _K)], idx_s[nb],
                                     sem_i)
                    pltpu.async_copy(dst_h.at[pl.ds(nxt, _K)], idx_d[nb],
                                     sem_i)

                @pl.loop(0, _K - 2, step=2)
                def _pair(g):
                    pltpu.make_async_copy(
                        h_h.at[idx_s[b].at[g]], rows_a, sem).wait()
                    pltpu.async_copy(h_h.at[idx_s[b].at[g + 1]], rows_b, sem)
                    pltpu.sync_copy(rows_a, acc.at[idx_d[b].at[g]], add=True)
                    pltpu.make_async_copy(
                        h_h.at[idx_s[b].at[g + 1]], rows_b, sem).wait()
                    pltpu.async_copy(h_h.at[idx_s[b].at[g + 2]], rows_a, sem)
                    pltpu.sync_copy(rows_b, acc.at[idx_d[b].at[g + 1]],
                                    add=True)

                if not last:
                    pltpu.make_async_copy(
                        src_h.at[pl.ds(nxt, _K)], idx_s[nb], sem_i).wait()
                    pltpu.make_async_copy(
                        dst_h.at[pl.ds(nxt, _K)], idx_d[nb], sem_i).wait()
                pltpu.make_async_copy(
                    h_h.at[idx_s[b].at[_K - 2]], rows_a, sem).wait()
                pltpu.async_copy(h_h.at[idx_s[b].at[_K - 1]], rows_b, sem)
                pltpu.sync_copy(rows_a, acc.at[idx_d[b].at[_K - 2]], add=True)
                pltpu.make_async_copy(
                    h_h.at[idx_s[b].at[_K - 1]], rows_b, sem).wait()
                if not last:
                    pltpu.async_copy(h_h.at[idx_s[nb].at[0]], rows_a, sem)
                pltpu.sync_copy(rows_b, acc.at[idx_d[b].at[_K - 1]], add=True)

        @pl.when(c == 0)
        def _core0():
            run(s * g0, g0 // _K)

        @pl.when(c == 1)
        def _core1():
            run(_NS * g0 + s * g1, g1 // _K)

        plsc.subcore_barrier()
        ob = c * n_pad + s * rows_per_tile
        pltpu.sync_copy(acc.at[pl.ds(s * rows_per_tile, rows_per_tile)],
                        out_h.at[pl.ds(ob, rows_per_tile)])

    return pl.kernel(
        body,
        out_type=jax.ShapeDtypeStruct((_NC * n_pad, d), jnp.float32),
        mesh=mesh,
        scratch_types=[
            pltpu.VMEM((_K, _GRP), jnp.int32),
            pltpu.VMEM((_K, _GRP), jnp.int32),
            pltpu.VMEM((_K, _GRP), jnp.int32),
            pltpu.VMEM((_K, _GRP), jnp.int32),
            pltpu.VMEM((_GRP, d), jnp.float32),
            pltpu.VMEM((_GRP, d), jnp.float32),
            pltpu.VMEM((_ZR, d), jnp.float32),
            pltpu.VMEM_SHARED((n_pad, d), jnp.float32),
            pltpu.SemaphoreType.DMA,
            pltpu.SemaphoreType.DMA,
        ],
    )


@functools.lru_cache(maxsize=None)
def _sc_cnt(n_pad, e_pad, r):
    g_per_w = e_pad // (_NW * _GRP)
    cnt_sz = n_pad * r
    cnt_per_tile = cnt_sz // _NS
    mesh = plsc.VectorSubcoreMesh(core_axis_name="c", subcore_axis_name="s")

    def body(dr_h, zf_h, cnt_h, idx_r, ones_v, acc_f):
        c = lax.axis_index("c")
        s = lax.axis_index("s")
        wid = c * _NS + s

        zbf = s * cnt_per_tile
        pltpu.sync_copy(zf_h.at[pl.ds(zbf, cnt_per_tile)],
                        acc_f.at[pl.ds(zbf, cnt_per_tile)])
        gb = wid * g_per_w
        pltpu.sync_copy(dr_h.at[pl.ds(gb, g_per_w)], idx_r)
        for j in range(_GRP // 16):
            ones_v[pl.ds(j * 16, 16)] = jnp.full((16,), 1.0, jnp.float32)
        plsc.subcore_barrier()

        @pl.loop(0, g_per_w)
        def _edge_group(g):
            pltpu.sync_copy(ones_v, acc_f.at[idx_r.at[g]], add=True)

        plsc.subcore_barrier()
        obf = c * cnt_sz + s * cnt_per_tile
        pltpu.sync_copy(acc_f.at[pl.ds(s * cnt_per_tile, cnt_per_tile)],
                        cnt_h.at[pl.ds(obf, cnt_per_tile)])

    return pl.kernel(
        body,
        out_type=jax.ShapeDtypeStruct((_NC * cnt_sz,), jnp.float32),
        mesh=mesh,
        scratch_types=[
            pltpu.VMEM((g_per_w, _GRP), jnp.int32),
            pltpu.VMEM((_GRP,), jnp.float32),
            pltpu.VMEM_SHARED((cnt_sz,), jnp.float32),
        ],
    )


@functools.lru_cache(maxsize=None)
def _tc_self(n_pad, d, blk=512):
    def body(h_ref, ws_ref, bs_ref, out_ref):
        out_ref[...] = lax.dot_general(
            h_ref[...], ws_ref[...], (((1,), (1,)), ((), ())),
            preferred_element_type=jnp.float32) + bs_ref[...]

    row_spec = pl.BlockSpec((blk, d), lambda i: (i, 0))
    return pl.pallas_call(
        body,
        grid=(n_pad // blk,),
        in_specs=[row_spec,
                  pl.BlockSpec((d, d), lambda i: (0, 0)),
                  pl.BlockSpec((1, d), lambda i: (0, 0))],
        out_specs=row_spec,
        out_shape=jax.ShapeDtypeStruct((n_pad, d), jnp.float32),
    )


@functools.lru_cache(maxsize=None)
def _tc_layer(n_pad, d, r, out_rows, blk=512):
    def body(selfv_ref, s0_ref, s1_ref, c0_ref, c1_ref, rel_ref,
             wn_ref, bn_ref, g_ref, b_ref, out_ref):
        cnt = c0_ref[...] + c1_ref[...]
        deg = jnp.sum(cnt, axis=1, keepdims=True)
        has_in = deg > 0.0
        denom = jnp.where(has_in, deg, 1.0)
        summed = (s0_ref[...] + s1_ref[...]
                  + jnp.dot(cnt, rel_ref[...],
                            preferred_element_type=jnp.float32))
        agg = summed / denom
        neigh = lax.dot_general(agg, wn_ref[...], (((1,), (1,)), ((), ())),
                                preferred_element_type=jnp.float32) + bn_ref[...]
        neigh = jnp.where(has_in, neigh, 0.0)
        pre = selfv_ref[...] + neigh
        mu = jnp.mean(pre, axis=1, keepdims=True)
        cent = pre - mu
        var = jnp.mean(cent * cent, axis=1, keepdims=True)
        normed = cent * lax.rsqrt(var + 1e-5)
        out_ref[...] = jnp.maximum(normed * g_ref[...] + b_ref[...], 0.0)

    nb = n_pad // blk
    row_spec = pl.BlockSpec((blk, d), lambda i: (i, 0))
    return pl.pallas_call(
        body,
        grid=(nb,),
        in_specs=[
            row_spec,
            pl.BlockSpec((blk, d), lambda i: (i, 0)),
            pl.BlockSpec((blk, d), lambda i: (i + nb, 0)),
            pl.BlockSpec((blk, r), lambda i: (i, 0)),
            pl.BlockSpec((blk, r), lambda i: (i + nb, 0)),
            pl.BlockSpec((r, d), lambda i: (0, 0)),
            pl.BlockSpec((d, d), lambda i: (0, 0)),
            pl.BlockSpec((1, d), lambda i: (0, 0)),
            pl.BlockSpec((1, d), lambda i: (0, 0)),
            pl.BlockSpec((1, d), lambda i: (0, 0)),
        ],
        out_specs=row_spec,
        out_shape=jax.ShapeDtypeStruct((out_rows, d), jnp.float32),
    )


def kernel(x, edge_index, edge_rel, rel_emb, W_neigh, b_neigh,
           W_self, b_self, ln_g, ln_b):
    n, d = x.shape
    e = edge_index.shape[1]
    r = rel_emb.shape[0]
    num_layers = W_neigh.shape[0]

    blk = 512
    n_pad = -(-n // blk) * blk
    g_pair = -(-e // (_NS * _GRP * _K)) * _K
    g0 = min(max(round(_SPLIT * g_pair / _K) * _K, _K), g_pair - _K)
    g1 = g_pair - g0
    e_pad = _NS * g_pair * _GRP

    src = edge_index[0].astype(jnp.int32)
    dst = edge_index[1].astype(jnp.int32)
    rel = edge_rel.astype(jnp.int32)
    pad_e = e_pad - e
    pad_i = jnp.arange(pad_e, dtype=jnp.int32)
    src_p = jnp.concatenate([src, pad_i % n]).reshape(-1, _GRP)
    dst_pad = jnp.concatenate([dst, n + pad_i % (n_pad - n)])
    rel_pad = jnp.concatenate([rel, jnp.zeros((pad_e,), jnp.int32)])
    dst_p = dst_pad.reshape(-1, _GRP)
    dr_p = (dst_pad * r + rel_pad).reshape(-1, _GRP)
    zf = jnp.zeros((n_pad * r,), jnp.float32)

    h = jnp.pad(x, ((0, n_pad - n), (0, 0)))

    spmm = _sc_spmm(n_pad, e_pad, d, g0, g1)
    cntk = _sc_cnt(n_pad, e_pad, r)

    cnt2 = cntk(dr_p, zf).reshape(_NC * n_pad, r)
    for l in range(num_layers):
        s2 = spmm(src_p, dst_p, h)
        selfv = _tc_self(n_pad, d, blk)(
            h, W_self[l], b_self[l].reshape(1, d))
        out_rows = n if l == num_layers - 1 else n_pad
        h = _tc_layer(n_pad, d, r, out_rows, blk)(
            selfv, s2, s2, cnt2, cnt2, rel_emb,
            W_neigh[l], b_neigh[l].reshape(1, d),
            ln_g[l].reshape(1, d), ln_b[l].reshape(1, d))
    return h


# speedup vs baseline: 1.0028x; 1.0028x over previous
import functools

import jax
import jax.numpy as jnp
from jax import lax
from jax.experimental import pallas as pl
from jax.experimental.pallas import tpu as pltpu
from jax.experimental.pallas import tpu_sc as plsc

_NC = 2
_NS = 16
_NW = _NC * _NS
_GRP = 128
_K = 8
_ZR = 64
_SPLIT = 0.5


@functools.lru_cache(maxsize=None)
def _sc_spmm(n_pad, e_pad, d, g0, g1):
    assert e_pad == _NS * (g0 + g1) * _GRP
    assert g0 % _K == 0 and g1 % _K == 0
    rows_per_tile = n_pad // _NS
    mesh = plsc.VectorSubcoreMesh(core_axis_name="c", subcore_axis_name="s")

    def body(src_h, dst_h, h_h, out_h,
             idx_s0, idx_s1, idx_d0, idx_d1, rows_a, rows_b, zbuf, acc,
             sem, sem_i):
        c = lax.axis_index("c")
        s = lax.axis_index("s")
        idx_s = [idx_s0, idx_s1]
        idx_d = [idx_d0, idx_d1]

        @pl.loop(0, _ZR)
        def _zfill(i):
            for j in range(d // 16):
                zbuf[i, pl.ds(j * 16, 16)] = jnp.zeros((16,), jnp.float32)
        zb = s * rows_per_tile

        @pl.loop(0, rows_per_tile // _ZR)
        def _zcopy(i):
            pltpu.sync_copy(zbuf, acc.at[pl.ds(zb + i * _ZR, _ZR)])

        plsc.subcore_barrier()

        def run(gbase, n_chunks):
            pltpu.sync_copy(src_h.at[pl.ds(gbase, _K)], idx_s[0])
            pltpu.sync_copy(dst_h.at[pl.ds(gbase, _K)], idx_d[0])
            pltpu.async_copy(h_h.at[idx_s[0].at[0]], rows_a, sem)

            for i in range(n_chunks):
                b, nb = i % 2, (i + 1) % 2
                last = i + 1 == n_chunks
                if not last:
                    nxt = gbase + (i + 1) * _K
                    pltpu.async_copy(src_h.at[pl.ds(nxt, _K)], idx_s[nb],
                                     sem_i)
                    pltpu.async_copy(dst_h.at[pl.ds(nxt, _K)], idx_d[nb],
                                     sem_i)

                @pl.loop(0, _K - 2, step=2)
                def _pair(g):
                    pltpu.make_async_copy(
                        h_h.at[idx_s[b].at[g]], rows_a, sem).wait()
                    pltpu.async_copy(h_h.at[idx_s[b].at[g + 1]], rows_b, sem)
                    pltpu.sync_copy(rows_a, acc.at[idx_d[b].at[g]], add=True)
                    pltpu.make_async_copy(
                        h_h.at[idx_s[b].at[g + 1]], rows_b, sem).wait()
                    pltpu.async_copy(h_h.at[idx_s[b].at[g + 2]], rows_a, sem)
                    pltpu.sync_copy(rows_b, acc.at[idx_d[b].at[g + 1]],
                                    add=True)

                if not last:
                    pltpu.make_async_copy(
                        src_h.at[pl.ds(nxt, _K)], idx_s[nb], sem_i).wait()
                    pltpu.make_async_copy(
                        dst_h.at[pl.ds(nxt, _K)], idx_d[nb], sem_i).wait()
                pltpu.make_async_copy(
                    h_h.at[idx_s[b].at[_K - 2]], rows_a, sem).wait()
                pltpu.async_copy(h_h.at[idx_s[b].at[_K - 1]], rows_b, sem)
                pltpu.sync_copy(rows_a, acc.at[idx_d[b].at[_K - 2]], add=True)
                pltpu.make_async_copy(
                    h_h.at[idx_s[b].at[_K - 1]], rows_b, sem).wait()
                if not last:
                    pltpu.async_copy(h_h.at[idx_s[nb].at[0]], rows_a, sem)
                pltpu.sync_copy(rows_b, acc.at[idx_d[b].at[_K - 1]], add=True)

        @pl.when(c == 0)
        def _core0():
            run(s * g0, g0 // _K)

        @pl.when(c == 1)
        def _core1():
            run(_NS * g0 + s * g1, g1 // _K)

        plsc.subcore_barrier()
        ob = c * n_pad + s * rows_per_tile
        pltpu.sync_copy(acc.at[pl.ds(s * rows_per_tile, rows_per_tile)],
                        out_h.at[pl.ds(ob, rows_per_tile)])

    return pl.kernel(
        body,
        out_type=jax.ShapeDtypeStruct((_NC * n_pad, d), jnp.float32),
        mesh=mesh,
        scratch_types=[
            pltpu.VMEM((_K, _GRP), jnp.int32),
            pltpu.VMEM((_K, _GRP), jnp.int32),
            pltpu.VMEM((_K, _GRP), jnp.int32),
            pltpu.VMEM((_K, _GRP), jnp.int32),
            pltpu.VMEM((_GRP, d), jnp.float32),
            pltpu.VMEM((_GRP, d), jnp.float32),
            pltpu.VMEM((_ZR, d), jnp.float32),
            pltpu.VMEM_SHARED((n_pad, d), jnp.float32),
            pltpu.SemaphoreType.DMA,
            pltpu.SemaphoreType.DMA,
        ],
    )


@functools.lru_cache(maxsize=None)
def _sc_cnt(n_pad, e_pad, r):
    g_per_w = e_pad // (_NW * _GRP)
    cnt_sz = n_pad * r
    cnt_per_tile = cnt_sz // _NS
    mesh = plsc.VectorSubcoreMesh(core_axis_name="c", subcore_axis_name="s")

    def body(dr_h, zf_h, cnt_h, idx_r, ones_v, acc_f):
        c = lax.axis_index("c")
        s = lax.axis_index("s")
        wid = c * _NS + s

        zbf = s * cnt_per_tile
        pltpu.sync_copy(zf_h.at[pl.ds(zbf, cnt_per_tile)],
                        acc_f.at[pl.ds(zbf, cnt_per_tile)])
        gb = wid * g_per_w
        pltpu.sync_copy(dr_h.at[pl.ds(gb, g_per_w)], idx_r)
        for j in range(_GRP // 16):
            ones_v[pl.ds(j * 16, 16)] = jnp.full((16,), 1.0, jnp.float32)
        plsc.subcore_barrier()

        @pl.loop(0, g_per_w)
        def _edge_group(g):
            pltpu.sync_copy(ones_v, acc_f.at[idx_r.at[g]], add=True)

        plsc.subcore_barrier()
        obf = c * cnt_sz + s * cnt_per_tile
        pltpu.sync_copy(acc_f.at[pl.ds(s * cnt_per_tile, cnt_per_tile)],
                        cnt_h.at[pl.ds(obf, cnt_per_tile)])

    return pl.kernel(
        body,
        out_type=jax.ShapeDtypeStruct((_NC * cnt_sz,), jnp.float32),
        mesh=mesh,
        scratch_types=[
            pltpu.VMEM((g_per_w, _GRP), jnp.int32),
            pltpu.VMEM((_GRP,), jnp.float32),
            pltpu.VMEM_SHARED((cnt_sz,), jnp.float32),
        ],
    )


@functools.lru_cache(maxsize=None)
def _tc_layer(n_pad, d, r, out_rows, blk=512):
    def body(h_ref, s0_ref, s1_ref, c0_ref, c1_ref, rel_ref,
             wn_ref, bn_ref, ws_ref, bs_ref, g_ref, b_ref, out_ref):
        cnt = c0_ref[...] + c1_ref[...]
        deg = jnp.sum(cnt, axis=1, keepdims=True)
        has_in = deg > 0.0
        denom = jnp.where(has_in, deg, 1.0)
        summed = (s0_ref[...] + s1_ref[...]
                  + jnp.dot(cnt, rel_ref[...],
                            preferred_element_type=jnp.float32))
        agg = summed / denom
        neigh = lax.dot_general(agg, wn_ref[...], (((1,), (1,)), ((), ())),
                                preferred_element_type=jnp.float32) + bn_ref[...]
        neigh = jnp.where(has_in, neigh, 0.0)
        selfv = lax.dot_general(h_ref[...], ws_ref[...], (((1,), (1,)), ((), ())),
                                preferred_element_type=jnp.float32) + bs_ref[...]
        pre = selfv + neigh
        mu = jnp.mean(pre, axis=1, keepdims=True)
        cent = pre - mu
        var = jnp.mean(cent * cent, axis=1, keepdims=True)
        normed = cent * lax.rsqrt(var + 1e-5)
        out_ref[...] = jnp.maximum(normed * g_ref[...] + b_ref[...], 0.0)

    nb = n_pad // blk
    row_spec = pl.BlockSpec((blk, d), lambda i: (i, 0))
    return pl.pallas_call(
        body,
        grid=(nb,),
        in_specs=[
            row_spec,
            pl.BlockSpec((blk, d), lambda i: (i, 0)),
            pl.BlockSpec((blk, d), lambda i: (i + nb, 0)),
            pl.BlockSpec((blk, r), lambda i: (i, 0)),
            pl.BlockSpec((blk, r), lambda i: (i + nb, 0)),
            pl.BlockSpec((r, d), lambda i: (0, 0)),
            pl.BlockSpec((d, d), lambda i: (0, 0)),
            pl.BlockSpec((1, d), lambda i: (0, 0)),
            pl.BlockSpec((d, d), lambda i: (0, 0)),
            pl.BlockSpec((1, d), lambda i: (0, 0)),
            pl.BlockSpec((1, d), lambda i: (0, 0)),
            pl.BlockSpec((1, d), lambda i: (0, 0)),
        ],
        out_specs=row_spec,
        out_shape=jax.ShapeDtypeStruct((out_rows, d), jnp.float32),
    )


def kernel(x, edge_index, edge_rel, rel_emb, W_neigh, b_neigh,
           W_self, b_self, ln_g, ln_b):
    n, d = x.shape
    e = edge_index.shape[1]
    r = rel_emb.shape[0]
    num_layers = W_neigh.shape[0]

    blk = 512
    n_pad = -(-n // blk) * blk
    g_pair = -(-e // (_NS * _GRP * _K)) * _K
    g0 = min(max(round(_SPLIT * g_pair / _K) * _K, _K), g_pair - _K)
    g1 = g_pair - g0
    e_pad = _NS * g_pair * _GRP

    src = edge_index[0].astype(jnp.int32)
    dst = edge_index[1].astype(jnp.int32)
    rel = edge_rel.astype(jnp.int32)
    pad_e = e_pad - e
    pad_i = jnp.arange(pad_e, dtype=jnp.int32)
    src_p = jnp.concatenate([src, pad_i % n]).reshape(-1, _GRP)
    dst_pad = jnp.concatenate([dst, n + pad_i % (n_pad - n)])
    rel_pad = jnp.concatenate([rel, jnp.zeros((pad_e,), jnp.int32)])
    dst_p = dst_pad.reshape(-1, _GRP)
    dr_p = (dst_pad * r + rel_pad).reshape(-1, _GRP)
    zf = jnp.zeros((n_pad * r,), jnp.float32)

    h = jnp.pad(x, ((0, n_pad - n), (0, 0)))

    spmm = _sc_spmm(n_pad, e_pad, d, g0, g1)
    cntk = _sc_cnt(n_pad, e_pad, r)

    cnt2 = cntk(dr_p, zf).reshape(_NC * n_pad, r)
    for l in range(num_layers):
        s2 = spmm(src_p, dst_p, h)
        out_rows = n if l == num_layers - 1 else n_pad
        h = _tc_layer(n_pad, d, r, out_rows, blk)(
            h, s2, s2, cnt2, cnt2, rel_emb,
            W_neigh[l], b_neigh[l].reshape(1, d),
            W_self[l], b_self[l].reshape(1, d),
            ln_g[l].reshape(1, d), ln_b[l].reshape(1, d))
    return h
